# baseline (speedup 1.0000x reference)
import functools

import jax
import jax.numpy as jnp
from jax import lax
from jax.experimental import pallas as pl
from jax.experimental.pallas import tpu as pltpu
from jax.experimental.pallas import tpu_sc as plsc

N = 320000
D = 128
B = 1024
NC = 2
NS = 16
NW = NC * NS
RPW = N // NW
CHUNK = 40
CPW = RPW // CHUNK
NBUF = 10
SPB = CPW // NBUF
BPS = B // NS


def _sc_segment_sum(rows_hbm_arr, ids3):
  mesh = plsc.VectorSubcoreMesh(core_axis_name="c", subcore_axis_name="s")

  scratch = [pltpu.VMEM((CPW, CHUNK), jnp.int32)]
  scratch += [pltpu.VMEM((CHUNK, D), jnp.float32) for _ in range(NBUF)]
  scratch += [pltpu.VMEM((BPS, D), jnp.float32)]
  scratch += [pltpu.VMEM_SHARED((B, D), jnp.float32)]
  scratch += [pltpu.SemaphoreType.DMA for _ in range(2 * NBUF)]

  @functools.partial(
      pl.kernel,
      mesh=mesh,
      out_type=jax.ShapeDtypeStruct((NC, B, D), jnp.float32),
      scratch_types=scratch,
  )
  def k(rows_hbm, ids_hbm, out_hbm, ids_v, *rest):
    bufs = rest[:NBUF]
    tmp_v = rest[NBUF]
    acc_sh = rest[NBUF + 1]
    gsems = rest[NBUF + 2:2 * NBUF + 2]
    ssems = rest[2 * NBUF + 2:]
    cid = lax.axis_index("c")
    sid = lax.axis_index("s")
    wid = cid * NS + sid

    def zrow(r, carry):
      for c in range(D // 16):
        tmp_v[r, pl.ds(c * 16, 16)] = jnp.zeros((16,), jnp.float32)
      return carry
    lax.fori_loop(0, BPS, zrow, 0)
    pltpu.sync_copy(tmp_v, acc_sh.at[pl.ds(sid * BPS, BPS)])

    pltpu.sync_copy(ids_hbm.at[wid], ids_v)
    plsc.subcore_barrier()

    base = wid * RPW

    def g_start(ci, b):
      pltpu.make_async_copy(rows_hbm.at[pl.ds(base + ci * CHUNK, CHUNK)],
                            bufs[b], gsems[b]).start()

    def g_wait(b):
      pltpu.make_async_copy(rows_hbm.at[pl.ds(base, CHUNK)],
                            bufs[b], gsems[b]).wait()

    for b in range(NBUF):
      g_start(b, b)

    def outer(i, carry):
      c0 = i * NBUF
      scatters = []
      for b in range(NBUF):
        g_wait(b)
        scatters.append(
            pltpu.async_copy(bufs[b], acc_sh.at[ids_v.at[c0 + b]], ssems[b],
                             add=True))
      for b in range(NBUF):
        scatters[b].wait()
        nxt = c0 + NBUF + b

        @pl.when(nxt < CPW)
        def _():
          g_start(nxt, b)
      return carry

    lax.fori_loop(0, SPB, outer, 0)

    plsc.subcore_barrier()
    pltpu.sync_copy(acc_sh.at[pl.ds(sid * BPS, BPS)], tmp_v)
    pltpu.sync_copy(tmp_v, out_hbm.at[cid, pl.ds(sid * BPS, BPS)])

  return k(rows_hbm_arr, ids3)


def _tc_block_sums(rows):
  def body(x_ref, o_ref):
    o_ref[...] = jnp.sum(x_ref[...].reshape(800, 16, D), axis=1)

  return pl.pallas_call(
      body,
      grid=(25,),
      in_specs=[pl.BlockSpec((12800, D), lambda i: (i, 0))],
      out_specs=pl.BlockSpec((800, D), lambda i: (i, 0)),
      out_shape=jax.ShapeDtypeStruct((20000, D), jnp.float32),
  )(rows)


def _combine(partials, bsums):
  def add_body(a_ref, b_ref, c_ref, o_ref):
    o_ref[...] = a_ref[...] + b_ref[...] + 1e-30 * c_ref[...]

  return pl.pallas_call(
      add_body,
      out_shape=jax.ShapeDtypeStruct((B, D), jnp.float32),
  )(partials[0], partials[1], bsums[:B])


def kernel(outputs, atom_split):
  ids3 = atom_split.astype(jnp.int32).reshape(NW, CPW, CHUNK)
  partials = _sc_segment_sum(outputs, ids3)
  bsums = _tc_block_sums(outputs)
  return _combine(partials, bsums)


# speedup vs baseline: 1.3020x; 1.3020x over previous
import functools

import jax
import jax.numpy as jnp
from jax import lax
from jax.experimental import pallas as pl
from jax.experimental.pallas import tpu as pltpu
from jax.experimental.pallas import tpu_sc as plsc

N = 320000
D = 128
B = 1024
NC = 2
NS = 16
NW = NC * NS

TBLK = 16
TBPW = 400
M = TBLK * TBPW * NW
NBLK = M // TBLK
SCH_C = 100
CPB = TBPW // SCH_C

SRPW = (N - M) // NW
CHUNK = 40
CPW = SRPW // CHUNK
NBUF = 6
SPB = CPW // NBUF
BPS = B // NS

ACC_C = 1152
ZPT_C = ACC_C // NS


def _tc_block_sums(rows_tc):
  def body(x_ref, o_ref):
    o_ref[...] = jnp.sum(x_ref[...].reshape(800, TBLK, D), axis=1)

  return pl.pallas_call(
      body,
      grid=(16,),
      in_specs=[pl.BlockSpec((12800, D), lambda i: (i, 0))],
      out_specs=pl.BlockSpec((800, D), lambda i: (i, 0)),
      out_shape=jax.ShapeDtypeStruct((NBLK, D), jnp.float32),
  )(rows_tc)


def _sc_rowwise(rows_hbm_arr, sc_ids3, tc_ids3):
  mesh = plsc.VectorSubcoreMesh(core_axis_name="c", subcore_axis_name="s")

  scratch = [pltpu.VMEM((CPW, CHUNK), jnp.int32)]
  scratch += [pltpu.VMEM((CHUNK, D), jnp.float32) for _ in range(NBUF)]
  scratch += [
      pltpu.VMEM((TBPW, TBLK), jnp.int32),
      pltpu.VMEM((TBLK, D), jnp.float32),
      pltpu.VMEM((BPS, D), jnp.float32),
      pltpu.VMEM_SHARED((B, D), jnp.float32),
  ]
  scratch += [pltpu.SemaphoreType.DMA for _ in range(2 * NBUF)]

  @functools.partial(
      pl.kernel,
      mesh=mesh,
      out_type=jax.ShapeDtypeStruct((NC, B, D), jnp.float32),
      scratch_types=scratch,
  )
  def k(rows_hbm, ids_hbm, tcids_hbm, out_hbm, ids_v, *rest):
    bufs = rest[:NBUF]
    tcids_v, mbuf, tmp_v, acc_sh = rest[NBUF:NBUF + 4]
    gsems = rest[NBUF + 4:2 * NBUF + 4]
    ssems = rest[2 * NBUF + 4:]
    cid = lax.axis_index("c")
    sid = lax.axis_index("s")
    wid = cid * NS + sid

    def zrow(r, carry):
      for c in range(D // 16):
        tmp_v[r, pl.ds(c * 16, 16)] = jnp.zeros((16,), jnp.float32)
      return carry
    lax.fori_loop(0, BPS, zrow, 0)
    pltpu.sync_copy(tmp_v, acc_sh.at[pl.ds(sid * BPS, BPS)])

    pltpu.sync_copy(ids_hbm.at[wid], ids_v)
    pltpu.sync_copy(tcids_hbm.at[wid], tcids_v)
    plsc.subcore_barrier()

    base = M + wid * SRPW

    def g_start(ci, b):
      pltpu.make_async_copy(rows_hbm.at[pl.ds(base + ci * CHUNK, CHUNK)],
                            bufs[b], gsems[b]).start()

    def g_wait(b):
      pltpu.make_async_copy(rows_hbm.at[pl.ds(base, CHUNK)],
                            bufs[b], gsems[b]).wait()

    for b in range(NBUF):
      g_start(b, b)

    def outer(i, carry):
      c0 = i * NBUF
      scatters = []
      for b in range(NBUF):
        g_wait(b)
        scatters.append(
            pltpu.async_copy(bufs[b], acc_sh.at[ids_v.at[c0 + b]], ssems[b],
                             add=True))
      for b in range(NBUF):
        scatters[b].wait()
        nxt = c0 + NBUF + b

        @pl.when(nxt < CPW)
        def _():
          g_start(nxt, b)
      return carry

    lax.fori_loop(0, SPB, outer, 0)

    tbase = wid * TBPW

    def mixed(j, carry):
      idrow = tcids_v[j, :]

      @pl.when(idrow[0] != idrow[TBLK - 1])
      def _():
        pltpu.sync_copy(rows_hbm.at[pl.ds((tbase + j) * TBLK, TBLK)], mbuf)
        pltpu.sync_copy(mbuf, acc_sh.at[tcids_v.at[j]], add=True)
      return carry

    lax.fori_loop(0, TBPW, mixed, 0)

    plsc.subcore_barrier()
    pltpu.sync_copy(acc_sh.at[pl.ds(sid * BPS, BPS)], tmp_v)
    pltpu.sync_copy(tmp_v, out_hbm.at[cid, pl.ds(sid * BPS, BPS)])

  return k(rows_hbm_arr, sc_ids3, tc_ids3)


def _sc_bsum_scatter(bsums, blk_idx3):
  mesh = plsc.VectorSubcoreMesh(core_axis_name="c", subcore_axis_name="s")

  scratch = [
      pltpu.VMEM((CPB, SCH_C), jnp.int32),
      pltpu.VMEM((2 * SCH_C, D), jnp.float32),
      pltpu.VMEM((ZPT_C, D), jnp.float32),
      pltpu.VMEM_SHARED((ACC_C, D), jnp.float32),
      pltpu.SemaphoreType.DMA,
  ]

  @functools.partial(
      pl.kernel,
      mesh=mesh,
      out_type=jax.ShapeDtypeStruct((NC, B, D), jnp.float32),
      scratch_types=scratch,
  )
  def k(bs_hbm, idx_hbm, out_hbm, idx_v, bs_v, tmp_v, acc_sh, sem):
    cid = lax.axis_index("c")
    sid = lax.axis_index("s")
    wid = cid * NS + sid

    def zrow(r, carry):
      for c in range(D // 16):
        tmp_v[r, pl.ds(c * 16, 16)] = jnp.zeros((16,), jnp.float32)
      return carry
    lax.fori_loop(0, ZPT_C, zrow, 0)
    pltpu.sync_copy(tmp_v, acc_sh.at[pl.ds(sid * ZPT_C, ZPT_C)])

    pltpu.sync_copy(idx_hbm.at[wid], idx_v)
    plsc.subcore_barrier()

    for r in range(CPB // 2):
      pltpu.sync_copy(
          bs_hbm.at[pl.ds(wid * TBPW + r * 2 * SCH_C, 2 * SCH_C)], bs_v)
      scatters = []
      for p in range(2):
        scatters.append(
            pltpu.async_copy(bs_v.at[pl.ds(p * SCH_C, SCH_C)],
                             acc_sh.at[idx_v.at[r * 2 + p]], sem, add=True))
      for s in scatters:
        s.wait()

    plsc.subcore_barrier()
    pltpu.sync_copy(acc_sh.at[pl.ds(sid * BPS, BPS)],
                    tmp_v.at[pl.ds(0, BPS)])
    pltpu.sync_copy(tmp_v.at[pl.ds(0, BPS)],
                    out_hbm.at[cid, pl.ds(sid * BPS, BPS)])

  return k(bsums, blk_idx3)


def _combine4(b_parts, c_parts):
  def add_body(a_ref, b_ref, c_ref, d_ref, o_ref):
    o_ref[...] = a_ref[...] + b_ref[...] + c_ref[...] + d_ref[...]

  return pl.pallas_call(
      add_body,
      out_shape=jax.ShapeDtypeStruct((B, D), jnp.float32),
  )(b_parts[0], b_parts[1], c_parts[0], c_parts[1])


def kernel(outputs, atom_split):
  ids = atom_split.astype(jnp.int32)
  sc_ids3 = ids[M:].reshape(NW, CPW, CHUNK)
  tc_ids3 = ids[:M].reshape(NW, TBPW, TBLK)
  first = tc_ids3[:, :, 0]
  last = tc_ids3[:, :, TBLK - 1]
  blk_idx3 = jnp.where(first == last, first, B).reshape(NW, CPB, SCH_C)

  bsums = _tc_block_sums(outputs)
  b_parts = _sc_rowwise(outputs, sc_ids3, tc_ids3)
  c_parts = _sc_bsum_scatter(bsums, blk_idx3)
  return _combine4(b_parts, c_parts)


# speedup vs baseline: 1.3546x; 1.0404x over previous
import functools

import jax
import jax.numpy as jnp
from jax import lax
from jax.experimental import pallas as pl
from jax.experimental.pallas import tpu as pltpu
from jax.experimental.pallas import tpu_sc as plsc

N = 320000
D = 128
B = 1024
NC = 2
NS = 16
NW = NC * NS

TBLK = 16
TBPW = 480
M = TBLK * TBPW * NW
NBLK = M // TBLK
SCH_C = 120
CPB = TBPW // SCH_C

SRPW = (N - M) // NW
CHUNK = 40
CPW = SRPW // CHUNK
NBUF = 8
SPB = -(-CPW // NBUF)
BPS = B // NS

ACC_C = 1152
ZPT_C = ACC_C // NS


def _tc_block_sums(rows_tc):
  def body(x_ref, o_ref):
    o_ref[...] = jnp.sum(x_ref[...].reshape(800, TBLK, D), axis=1)

  return pl.pallas_call(
      body,
      grid=(16,),
      in_specs=[pl.BlockSpec((12800, D), lambda i: (i, 0))],
      out_specs=pl.BlockSpec((800, D), lambda i: (i, 0)),
      out_shape=jax.ShapeDtypeStruct((NBLK, D), jnp.float32),
  )(rows_tc)


def _sc_rowwise(rows_hbm_arr, sc_ids3, tc_ids3):
  mesh = plsc.VectorSubcoreMesh(core_axis_name="c", subcore_axis_name="s")

  scratch = [pltpu.VMEM((CPW, CHUNK), jnp.int32)]
  scratch += [pltpu.VMEM((CHUNK, D), jnp.float32) for _ in range(NBUF)]
  scratch += [
      pltpu.VMEM((TBPW, TBLK), jnp.int32),
      pltpu.VMEM((TBLK, D), jnp.float32),
      pltpu.VMEM((BPS, D), jnp.float32),
      pltpu.VMEM_SHARED((B, D), jnp.float32),
  ]
  scratch += [pltpu.SemaphoreType.DMA for _ in range(2 * NBUF)]

  @functools.partial(
      pl.kernel,
      mesh=mesh,
      out_type=jax.ShapeDtypeStruct((NC, B, D), jnp.float32),
      scratch_types=scratch,
  )
  def k(rows_hbm, ids_hbm, tcids_hbm, out_hbm, ids_v, *rest):
    bufs = rest[:NBUF]
    tcids_v, mbuf, tmp_v, acc_sh = rest[NBUF:NBUF + 4]
    gsems = rest[NBUF + 4:2 * NBUF + 4]
    ssems = rest[2 * NBUF + 4:]
    cid = lax.axis_index("c")
    sid = lax.axis_index("s")
    wid = cid * NS + sid

    def zrow(r, carry):
      for c in range(D // 16):
        tmp_v[r, pl.ds(c * 16, 16)] = jnp.zeros((16,), jnp.float32)
      return carry
    lax.fori_loop(0, BPS, zrow, 0)
    pltpu.sync_copy(tmp_v, acc_sh.at[pl.ds(sid * BPS, BPS)])

    pltpu.sync_copy(ids_hbm.at[wid], ids_v)
    pltpu.sync_copy(tcids_hbm.at[wid], tcids_v)
    plsc.subcore_barrier()

    base = M + wid * SRPW

    def g_start(ci, b):
      pltpu.make_async_copy(rows_hbm.at[pl.ds(base + ci * CHUNK, CHUNK)],
                            bufs[b], gsems[b]).start()

    def g_wait(b):
      pltpu.make_async_copy(rows_hbm.at[pl.ds(base, CHUNK)],
                            bufs[b], gsems[b]).wait()

    for b in range(NBUF):
      g_start(b, b)

    def outer(i, carry):
      c0 = i * NBUF
      for b in range(NBUF):
        @pl.when(c0 + b < CPW)
        def _():
          g_wait(b)
          pltpu.async_copy(bufs[b], acc_sh.at[ids_v.at[c0 + b]], ssems[b],
                           add=True)
      for b in range(NBUF):
        @pl.when(c0 + b < CPW)
        def _():
          pltpu.make_async_copy(bufs[b], acc_sh.at[ids_v.at[c0 + b]],
                                ssems[b]).wait()
        nxt = c0 + NBUF + b

        @pl.when(nxt < CPW)
        def _():
          g_start(nxt, b)
      return carry

    lax.fori_loop(0, SPB, outer, 0)

    tbase = wid * TBPW

    def mixed(j, carry):
      idrow = tcids_v[j, :]

      @pl.when(idrow[0] != idrow[TBLK - 1])
      def _():
        pltpu.sync_copy(rows_hbm.at[pl.ds((tbase + j) * TBLK, TBLK)], mbuf)
        pltpu.sync_copy(mbuf, acc_sh.at[tcids_v.at[j]], add=True)
      return carry

    lax.fori_loop(0, TBPW, mixed, 0)

    plsc.subcore_barrier()
    pltpu.sync_copy(acc_sh.at[pl.ds(sid * BPS, BPS)], tmp_v)
    pltpu.sync_copy(tmp_v, out_hbm.at[cid, pl.ds(sid * BPS, BPS)])

  return k(rows_hbm_arr, sc_ids3, tc_ids3)


def _sc_bsum_scatter(bsums, blk_idx3):
  mesh = plsc.VectorSubcoreMesh(core_axis_name="c", subcore_axis_name="s")

  scratch = [
      pltpu.VMEM((CPB, SCH_C), jnp.int32),
      pltpu.VMEM((2 * SCH_C, D), jnp.float32),
      pltpu.VMEM((ZPT_C, D), jnp.float32),
      pltpu.VMEM_SHARED((ACC_C, D), jnp.float32),
      pltpu.SemaphoreType.DMA,
  ]

  @functools.partial(
      pl.kernel,
      mesh=mesh,
      out_type=jax.ShapeDtypeStruct((NC, B, D), jnp.float32),
      scratch_types=scratch,
  )
  def k(bs_hbm, idx_hbm, out_hbm, idx_v, bs_v, tmp_v, acc_sh, sem):
    cid = lax.axis_index("c")
    sid = lax.axis_index("s")
    wid = cid * NS + sid

    def zrow(r, carry):
      for c in range(D // 16):
        tmp_v[r, pl.ds(c * 16, 16)] = jnp.zeros((16,), jnp.float32)
      return carry
    lax.fori_loop(0, ZPT_C, zrow, 0)
    pltpu.sync_copy(tmp_v, acc_sh.at[pl.ds(sid * ZPT_C, ZPT_C)])

    pltpu.sync_copy(idx_hbm.at[wid], idx_v)
    plsc.subcore_barrier()

    for r in range(CPB // 2):
      pltpu.sync_copy(
          bs_hbm.at[pl.ds(wid * TBPW + r * 2 * SCH_C, 2 * SCH_C)], bs_v)
      scatters = []
      for p in range(2):
        scatters.append(
            pltpu.async_copy(bs_v.at[pl.ds(p * SCH_C, SCH_C)],
                             acc_sh.at[idx_v.at[r * 2 + p]], sem, add=True))
      for s in scatters:
        s.wait()

    plsc.subcore_barrier()
    pltpu.sync_copy(acc_sh.at[pl.ds(sid * BPS, BPS)],
                    tmp_v.at[pl.ds(0, BPS)])
    pltpu.sync_copy(tmp_v.at[pl.ds(0, BPS)],
                    out_hbm.at[cid, pl.ds(sid * BPS, BPS)])

  return k(bsums, blk_idx3)


def _combine4(b_parts, c_parts):
  def add_body(a_ref, b_ref, c_ref, d_ref, o_ref):
    o_ref[...] = a_ref[...] + b_ref[...] + c_ref[...] + d_ref[...]

  return pl.pallas_call(
      add_body,
      out_shape=jax.ShapeDtypeStruct((B, D), jnp.float32),
  )(b_parts[0], b_parts[1], c_parts[0], c_parts[1])


def kernel(outputs, atom_split):
  ids = atom_split.astype(jnp.int32)
  sc_ids3 = ids[M:].reshape(NW, CPW, CHUNK)
  tc_ids3 = ids[:M].reshape(NW, TBPW, TBLK)
  first = tc_ids3[:, :, 0]
  last = tc_ids3[:, :, TBLK - 1]
  blk_idx3 = jnp.where(first == last, first, B).reshape(NW, CPB, SCH_C)

  bsums = _tc_block_sums(outputs)
  b_parts = _sc_rowwise(outputs, sc_ids3, tc_ids3)
  c_parts = _sc_bsum_scatter(bsums, blk_idx3)
  return _combine4(b_parts, c_parts)
